# Initial kernel scaffold; baseline (speedup 1.0000x reference)
import jax
import jax.numpy as jnp
from jax.experimental import pallas as pl


def kernel(x, edge_index, W, a):
    raise NotImplementedError("write your pallas kernel here")


# speedup vs baseline: 37.6997x; 37.6997x over previous
import functools

import jax
import jax.numpy as jnp
from jax import lax
from jax.experimental import pallas as pl
from jax.experimental.pallas import tpu as pltpu
from jax.experimental.pallas import tpu_sc as plsc

N_NODES = 2048
N_EDGES = 65536
IN_FEATURES = 256
HEADS = 8
OUT_FEATURES = 32
HF = HEADS * OUT_FEATURES

NC = 2
NS = 16
NW = NC * NS
EPW = N_EDGES // NW
CH = 128
NCH = EPW // CH

ROW_BLK = 256
GRID = N_NODES // ROW_BLK

_PREC = lax.Precision.HIGHEST


def _k1_body(x_ref, w_ref, acat_ref, r16_ref, g_ref, e_ref):
    h = lax.dot_general(x_ref[...], w_ref[...], (((1,), (0,)), ((), ())),
                        preferred_element_type=jnp.float32, precision=_PREC)
    s = lax.dot_general(h, acat_ref[...], (((1,), (0,)), ((), ())),
                        preferred_element_type=jnp.float32, precision=_PREC)
    e = jnp.exp(s)
    e_ref[...] = e
    edfull = lax.dot_general(e, r16_ref[...], (((1,), (0,)), ((), ())),
                             preferred_element_type=jnp.float32, precision=_PREC)
    g_ref[...] = h * edfull


_k1 = pl.pallas_call(
    _k1_body,
    grid=(GRID,),
    in_specs=[
        pl.BlockSpec((ROW_BLK, IN_FEATURES), lambda i: (i, 0)),
        pl.BlockSpec((IN_FEATURES, HF), lambda i: (0, 0)),
        pl.BlockSpec((HF, 16), lambda i: (0, 0)),
        pl.BlockSpec((16, HF), lambda i: (0, 0)),
    ],
    out_specs=[
        pl.BlockSpec((ROW_BLK, HF), lambda i: (i, 0)),
        pl.BlockSpec((ROW_BLK, 16), lambda i: (i, 0)),
    ],
    out_shape=[
        jax.ShapeDtypeStruct((N_NODES, HF), jnp.float32),
        jax.ShapeDtypeStruct((N_NODES, 16), jnp.float32),
    ],
    compiler_params=pltpu.CompilerParams(dimension_semantics=("parallel",)),
)


_sc_mesh = plsc.VectorSubcoreMesh(core_axis_name="c", subcore_axis_name="s")


@functools.partial(
    pl.kernel, mesh=_sc_mesh, out_type=(),
    scratch_types=[
        pltpu.VMEM((16, CH), jnp.int32),
        pltpu.VMEM((16, CH), jnp.int32),
        pltpu.VMEM((16, CH), jnp.int32),
        pltpu.VMEM((CH,), jnp.float32),
        pltpu.SemaphoreType.DMA,
    ],
    name="sc_scatter_mask",
)
def _k2_scatter_mask(row2_hbm, col2_hbm, m_ref, rowv, colv, keyv, onesv, sem):
    wid = lax.axis_index("s") * NC + lax.axis_index("c")
    pltpu.sync_copy(row2_hbm.at[pl.ds(wid * 16, 16)], rowv)
    pltpu.sync_copy(col2_hbm.at[pl.ds(wid * 16, 16)], colv)
    for l in range(CH // 16):
        onesv[pl.ds(l * 16, 16)] = jnp.ones((16,), jnp.float32)
    for j in range(NCH):
        for l in range(CH // 16):
            sl = pl.ds(l * 16, 16)
            keyv[j, sl] = rowv[j, sl] * N_NODES + colv[j, sl]
    copies = [pltpu.async_copy(onesv, m_ref.at[keyv.at[j]], sem)
              for j in range(NCH)]
    for cp in copies:
        cp.wait()


RPW = N_NODES // NW
NRANGE = NW
CAP = 128
SLOTS = NRANGE * NW * CAP
SENT = RPW


@functools.partial(
    pl.kernel, mesh=_sc_mesh,
    out_type=(
        jax.ShapeDtypeStruct((SLOTS, HF), jnp.float32),
        jax.ShapeDtypeStruct((SLOTS,), jnp.int32),
    ),
    scratch_types=[
        pltpu.VMEM((16, CH), jnp.int32),
        pltpu.VMEM((16, CH), jnp.int32),
        pltpu.VMEM((16, CH), jnp.int32),
        pltpu.VMEM((16, CH), jnp.int32),
        pltpu.VMEM((16, CH), jnp.int32),
        pltpu.VMEM((CH,), jnp.float32),
        pltpu.VMEM((NRANGE,), jnp.int32),
        pltpu.VMEM((CH,), jnp.int32),
        pltpu.VMEM((2, CH, HF), jnp.float32),
        pltpu.SemaphoreType.DMA,
        pltpu.SemaphoreType.DMA,
        pltpu.SemaphoreType.DMA,
    ],
    name="sc_scatter_csr",
    compiler_params=pltpu.CompilerParams(needs_layout_passes=False),
)
def _k3_scatter_csr(row2_hbm, col2_hbm, g_hbm, m_ref, val_ref, rid_ref,
                    rowv, colv, slotv, rlv, keyv, onesv, counters, sentv,
                    rows_v, sem_g, sem_s, sem_r):
    cid = lax.axis_index("c")
    sid = lax.axis_index("s")
    wid = sid * NC + cid
    pltpu.sync_copy(row2_hbm.at[pl.ds(wid * 16, 16)], rowv)
    pltpu.sync_copy(col2_hbm.at[pl.ds(wid * 16, 16)], colv)
    for l in range(NRANGE // 16):
        counters[pl.ds(l * 16, 16)] = jnp.zeros((16,), jnp.int32)
    for l in range(CH // 16):
        sentv[pl.ds(l * 16, 16)] = jnp.full((16,), SENT, jnp.int32)
    sent_cps = [pltpu.async_copy(
        sentv, rid_ref.at[pl.ds(rng * (NW * CAP) + wid * CAP, CAP)], sem_r)
        for rng in range(NRANGE)]
    for j in range(NCH):
        for l in range(CH // 16):
            sl = pl.ds(l * 16, 16)
            r16 = rowv[j, sl]
            rng16 = lax.shift_right_logical(r16, 6)
            base16 = plsc.load_gather(counters, [rng16])
            off16, last16 = plsc.scan_count(rng16)
            rank16 = base16 + off16 - 1
            plsc.store_scatter(counters, [rng16], rank16 + 1, mask=last16)
            rankc = jnp.minimum(rank16, CAP - 1)
            slotv[j, sl] = rng16 * (NW * CAP) + wid * CAP + rankc
            rlv[j, sl] = r16 - rng16 * RPW
            keyv[j, sl] = r16 * N_NODES + colv[j, sl]
    for l in range(CH // 16):
        onesv[pl.ds(l * 16, 16)] = jnp.ones((16,), jnp.float32)
    for cp in sent_cps:
        cp.wait()
    rid_cps = [pltpu.async_copy(rlv.at[j], rid_ref.at[slotv.at[j]], sem_r)
               for j in range(NCH)]
    m_cps = [pltpu.async_copy(onesv, m_ref.at[keyv.at[j]], sem_r)
             for j in range(NCH)]
    gcps, scps = [], []
    for j in range(NCH):
        if j >= 2:
            scps[j - 2].wait()
        gcps.append(pltpu.async_copy(
            g_hbm.at[colv.at[j]], rows_v.at[j % 2], sem_g))
        gcps[j].wait()
        scps.append(pltpu.async_copy(
            rows_v.at[j % 2], val_ref.at[slotv.at[j]], sem_s))
    for j in range(NCH - 2, NCH):
        scps[j].wait()
    for cp in rid_cps:
        cp.wait()
    for cp in m_cps:
        cp.wait()


def _k5_body(val_ref, ridl_ref, ridc_ref, out_ref):
    ridl = ridl_ref[0]
    ridc = ridc_ref[0]
    v = val_ref[...]
    vm = jnp.where(ridc < SENT, v, 0.0)
    iot = lax.broadcasted_iota(jnp.int32, (RPW, NW * CAP), 0)
    p = jnp.where(iot == ridl, 1.0, 0.0)
    out_ref[...] = lax.dot_general(
        p, vm, (((1,), (0,)), ((), ())),
        preferred_element_type=jnp.float32, precision=_PREC)


_k5 = pl.pallas_call(
    _k5_body,
    grid=(NRANGE,),
    in_specs=[
        pl.BlockSpec((NW * CAP, HF), lambda i: (i, 0)),
        pl.BlockSpec((1, 1, NW * CAP), lambda i: (i, 0, 0)),
        pl.BlockSpec((1, NW * CAP, 1), lambda i: (i, 0, 0)),
    ],
    out_specs=pl.BlockSpec((RPW, HF), lambda i: (i, 0)),
    out_shape=jax.ShapeDtypeStruct((N_NODES, HF), jnp.float32),
    compiler_params=pltpu.CompilerParams(dimension_semantics=("parallel",)),
)


def _k4_body(m_ref, ed1_ref, es_ref, agg_ref, r8_ref, out_ref):
    td = lax.dot_general(m_ref[...], ed1_ref[...], (((1,), (0,)), ((), ())),
                         preferred_element_type=jnp.float32, precision=_PREC)
    t = td[:, 0:8]
    d = td[:, 8:9]
    esv = es_ref[...]
    denom = esv * t + (float(N_NODES) - d)
    scale = esv / denom
    sf = lax.dot_general(scale, r8_ref[...], (((1,), (0,)), ((), ())),
                         preferred_element_type=jnp.float32, precision=_PREC)
    out_ref[...] = jnp.maximum(agg_ref[...] * sf, 0.0)


_k4 = pl.pallas_call(
    _k4_body,
    grid=(GRID,),
    in_specs=[
        pl.BlockSpec((ROW_BLK, N_NODES), lambda i: (i, 0)),
        pl.BlockSpec((N_NODES, 16), lambda i: (0, 0)),
        pl.BlockSpec((ROW_BLK, HEADS), lambda i: (i, 0)),
        pl.BlockSpec((ROW_BLK, HF), lambda i: (i, 0)),
        pl.BlockSpec((HEADS, HF), lambda i: (0, 0)),
    ],
    out_specs=pl.BlockSpec((ROW_BLK, HF), lambda i: (i, 0)),
    out_shape=jax.ShapeDtypeStruct((N_NODES, HF), jnp.float32),
    compiler_params=pltpu.CompilerParams(dimension_semantics=("parallel",)),
)


def kernel(x, edge_index, W, a):
    a0 = a[0]
    asrc = a0[:, :OUT_FEATURES]
    adst = a0[:, OUT_FEATURES:]
    eye8 = jnp.eye(HEADS, dtype=jnp.float32)
    acat = jnp.concatenate(
        [(eye8[:, None, :] * asrc[:, :, None]).reshape(HF, HEADS),
         (eye8[:, None, :] * adst[:, :, None]).reshape(HF, HEADS)], axis=1)
    r8 = jnp.repeat(eye8, OUT_FEATURES, axis=1)
    r16 = jnp.concatenate([jnp.zeros((8, HF), jnp.float32), r8], axis=0)

    row2 = edge_index[0].reshape(NW * 16, CH)
    col2 = edge_index[1].reshape(NW * 16, CH)

    g, expsc = _k1(x, W, acat, r16)

    m_ref = jax.new_ref(jnp.zeros((N_NODES * N_NODES,), jnp.float32))
    val, rid = _k3_scatter_csr(row2, col2, g, m_ref)
    M = m_ref[...].reshape(N_NODES, N_NODES)
    agg = _k5(val, rid.reshape(NRANGE, 1, NW * CAP),
              rid.reshape(NRANGE, NW * CAP, 1))

    es = expsc[:, :HEADS]
    ed1 = jnp.concatenate(
        [expsc[:, HEADS:], jnp.ones((N_NODES, 1), jnp.float32),
         jnp.zeros((N_NODES, 7), jnp.float32)], axis=1)

    return _k4(M, ed1, es, agg, r8)


# speedup vs baseline: 52.2975x; 1.3872x over previous
import functools

import jax
import jax.numpy as jnp
from jax import lax
from jax.experimental import pallas as pl
from jax.experimental.pallas import tpu as pltpu
from jax.experimental.pallas import tpu_sc as plsc

N_NODES = 2048
N_EDGES = 65536
IN_FEATURES = 256
HEADS = 8
OUT_FEATURES = 32
HF = HEADS * OUT_FEATURES

NC = 2
NS = 16
NW = NC * NS
EPW = N_EDGES // NW
CH = 128
NCH = EPW // CH

ROW_BLK = 256
GRID = N_NODES // ROW_BLK

_PREC = lax.Precision.HIGHEST


def _k1_body(x_ref, w_ref, acat_ref, r16_ref, g_ref, e_ref):
    h = lax.dot_general(x_ref[...], w_ref[...], (((1,), (0,)), ((), ())),
                        preferred_element_type=jnp.float32, precision=_PREC)
    s = lax.dot_general(h, acat_ref[...], (((1,), (0,)), ((), ())),
                        preferred_element_type=jnp.float32, precision=_PREC)
    e = jnp.exp(s)
    e_ref[...] = e
    edfull = lax.dot_general(e, r16_ref[...], (((1,), (0,)), ((), ())),
                             preferred_element_type=jnp.float32, precision=_PREC)
    g_ref[...] = h * edfull


_k1 = pl.pallas_call(
    _k1_body,
    grid=(GRID,),
    in_specs=[
        pl.BlockSpec((ROW_BLK, IN_FEATURES), lambda i: (i, 0)),
        pl.BlockSpec((IN_FEATURES, HF), lambda i: (0, 0)),
        pl.BlockSpec((HF, 16), lambda i: (0, 0)),
        pl.BlockSpec((16, HF), lambda i: (0, 0)),
    ],
    out_specs=[
        pl.BlockSpec((ROW_BLK, HF), lambda i: (i, 0)),
        pl.BlockSpec((ROW_BLK, 16), lambda i: (i, 0)),
    ],
    out_shape=[
        jax.ShapeDtypeStruct((N_NODES, HF), jnp.float32),
        jax.ShapeDtypeStruct((N_NODES, 16), jnp.float32),
    ],
    compiler_params=pltpu.CompilerParams(dimension_semantics=("parallel",)),
)


_sc_mesh = plsc.VectorSubcoreMesh(core_axis_name="c", subcore_axis_name="s")


@functools.partial(
    pl.kernel, mesh=_sc_mesh, out_type=(),
    scratch_types=[
        pltpu.VMEM((16, CH), jnp.int32),
        pltpu.VMEM((16, CH), jnp.int32),
        pltpu.VMEM((16, CH), jnp.int32),
        pltpu.VMEM((CH,), jnp.float32),
        pltpu.SemaphoreType.DMA,
    ],
    name="sc_scatter_mask",
)
def _k2_scatter_mask(row2_hbm, col2_hbm, m_ref, rowv, colv, keyv, onesv, sem):
    wid = lax.axis_index("s") * NC + lax.axis_index("c")
    pltpu.sync_copy(row2_hbm.at[pl.ds(wid * 16, 16)], rowv)
    pltpu.sync_copy(col2_hbm.at[pl.ds(wid * 16, 16)], colv)
    for l in range(CH // 16):
        onesv[pl.ds(l * 16, 16)] = jnp.ones((16,), jnp.float32)
    for j in range(NCH):
        for l in range(CH // 16):
            sl = pl.ds(l * 16, 16)
            keyv[j, sl] = rowv[j, sl] * N_NODES + colv[j, sl]
    copies = [pltpu.async_copy(onesv, m_ref.at[keyv.at[j]], sem)
              for j in range(NCH)]
    for cp in copies:
        cp.wait()


RPW = N_NODES // NW
NRANGE = NW
CAP = 128
SLOTS = NRANGE * NW * CAP
SENT = RPW


@functools.partial(
    pl.kernel, mesh=_sc_mesh,
    out_type=(
        jax.ShapeDtypeStruct((SLOTS, HF), jnp.float32),
        jax.ShapeDtypeStruct((SLOTS,), jnp.int32),
    ),
    scratch_types=[
        pltpu.VMEM((16, CH), jnp.int32),
        pltpu.VMEM((16, CH), jnp.int32),
        pltpu.VMEM((16, CH), jnp.int32),
        pltpu.VMEM((16, CH), jnp.int32),
        pltpu.VMEM((16, CH), jnp.int32),
        pltpu.VMEM((CH,), jnp.float32),
        pltpu.VMEM((NRANGE,), jnp.int32),
        pltpu.VMEM((CH,), jnp.int32),
        pltpu.VMEM((3, CH, HF), jnp.float32),
        pltpu.SemaphoreType.DMA,
        pltpu.SemaphoreType.DMA,
        pltpu.SemaphoreType.DMA,
    ],
    name="sc_scatter_csr",
    compiler_params=pltpu.CompilerParams(needs_layout_passes=False),
)
def _k3_scatter_csr(row2_hbm, col2_hbm, g_hbm, m_ref, val_ref, rid_ref,
                    rowv, colv, slotv, rlv, keyv, onesv, counters, sentv,
                    rows_v, sem_g, sem_s, sem_r):
    cid = lax.axis_index("c")
    sid = lax.axis_index("s")
    wid = sid * NC + cid
    pltpu.sync_copy(row2_hbm.at[pl.ds(wid * 16, 16)], rowv)
    pltpu.sync_copy(col2_hbm.at[pl.ds(wid * 16, 16)], colv)
    for l in range(NRANGE // 16):
        counters[pl.ds(l * 16, 16)] = jnp.zeros((16,), jnp.int32)
    for l in range(CH // 16):
        sentv[pl.ds(l * 16, 16)] = jnp.full((16,), SENT, jnp.int32)
    sent_cps = [pltpu.async_copy(
        sentv, rid_ref.at[pl.ds(rng * (NW * CAP) + wid * CAP, CAP)], sem_r)
        for rng in range(NRANGE)]
    for j in range(NCH):
        for l in range(CH // 16):
            sl = pl.ds(l * 16, 16)
            r16 = rowv[j, sl]
            rng16 = lax.shift_right_logical(r16, 6)
            base16 = plsc.load_gather(counters, [rng16])
            off16, last16 = plsc.scan_count(rng16)
            rank16 = base16 + off16 - 1
            plsc.store_scatter(counters, [rng16], rank16 + 1, mask=last16)
            rankc = jnp.minimum(rank16, CAP - 1)
            slotv[j, sl] = rng16 * (NW * CAP) + wid * CAP + rankc
            rlv[j, sl] = r16 - rng16 * RPW
            keyv[j, sl] = r16 * N_NODES + colv[j, sl]
    for l in range(CH // 16):
        onesv[pl.ds(l * 16, 16)] = jnp.ones((16,), jnp.float32)
    for cp in sent_cps:
        cp.wait()
    rid_cps = [pltpu.async_copy(rlv.at[j], rid_ref.at[slotv.at[j]], sem_r)
               for j in range(NCH)]
    m_cps = [pltpu.async_copy(onesv, m_ref.at[keyv.at[j]], sem_r)
             for j in range(NCH)]
    gcps, scps = [], []
    gcps.append(pltpu.async_copy(
        g_hbm.at[colv.at[0]], rows_v.at[0], sem_g))
    for j in range(NCH):
        nxt = j + 1
        if nxt < NCH:
            if nxt >= 3:
                scps[nxt - 3].wait()
            gcps.append(pltpu.async_copy(
                g_hbm.at[colv.at[nxt]], rows_v.at[nxt % 3], sem_g))
        gcps[j].wait()
        scps.append(pltpu.async_copy(
            rows_v.at[j % 3], val_ref.at[slotv.at[j]], sem_s))
    for j in range(NCH - 3, NCH):
        scps[j].wait()
    for cp in rid_cps:
        cp.wait()
    for cp in m_cps:
        cp.wait()


def _k5_body(val_ref, ridl_ref, out_ref):
    ridl = ridl_ref[0]
    v = val_ref[...]
    vm = jnp.where(jnp.abs(v) < 3e38, v, 0.0)
    iot = lax.broadcasted_iota(jnp.int32, (RPW, NW * CAP), 0)
    p = jnp.where(iot == ridl, 1.0, 0.0)
    out_ref[...] = lax.dot_general(
        p, vm, (((1,), (0,)), ((), ())),
        preferred_element_type=jnp.float32)


_k5 = pl.pallas_call(
    _k5_body,
    grid=(NRANGE,),
    in_specs=[
        pl.BlockSpec((NW * CAP, HF), lambda i: (i, 0)),
        pl.BlockSpec((1, 1, NW * CAP), lambda i: (i, 0, 0)),
    ],
    out_specs=pl.BlockSpec((RPW, HF), lambda i: (i, 0)),
    out_shape=jax.ShapeDtypeStruct((N_NODES, HF), jnp.float32),
    compiler_params=pltpu.CompilerParams(dimension_semantics=("parallel",)),
)


def _k4_body(m_ref, ed1_ref, es_ref, agg_ref, r8_ref, out_ref):
    td = lax.dot_general(m_ref[...], ed1_ref[...], (((1,), (0,)), ((), ())),
                         preferred_element_type=jnp.float32, precision=_PREC)
    t = td[:, 0:8]
    d = td[:, 8:9]
    esv = es_ref[...]
    denom = esv * t + (float(N_NODES) - d)
    scale = esv / denom
    sf = lax.dot_general(scale, r8_ref[...], (((1,), (0,)), ((), ())),
                         preferred_element_type=jnp.float32, precision=_PREC)
    out_ref[...] = jnp.maximum(agg_ref[...] * sf, 0.0)


_k4 = pl.pallas_call(
    _k4_body,
    grid=(GRID,),
    in_specs=[
        pl.BlockSpec((ROW_BLK, N_NODES), lambda i: (i, 0)),
        pl.BlockSpec((N_NODES, 16), lambda i: (0, 0)),
        pl.BlockSpec((ROW_BLK, HEADS), lambda i: (i, 0)),
        pl.BlockSpec((ROW_BLK, HF), lambda i: (i, 0)),
        pl.BlockSpec((HEADS, HF), lambda i: (0, 0)),
    ],
    out_specs=pl.BlockSpec((ROW_BLK, HF), lambda i: (i, 0)),
    out_shape=jax.ShapeDtypeStruct((N_NODES, HF), jnp.float32),
    compiler_params=pltpu.CompilerParams(dimension_semantics=("parallel",)),
)


def kernel(x, edge_index, W, a):
    a0 = a[0]
    asrc = a0[:, :OUT_FEATURES]
    adst = a0[:, OUT_FEATURES:]
    eye8 = jnp.eye(HEADS, dtype=jnp.float32)
    acat = jnp.concatenate(
        [(eye8[:, None, :] * asrc[:, :, None]).reshape(HF, HEADS),
         (eye8[:, None, :] * adst[:, :, None]).reshape(HF, HEADS)], axis=1)
    r8 = jnp.repeat(eye8, OUT_FEATURES, axis=1)
    r16 = jnp.concatenate([jnp.zeros((8, HF), jnp.float32), r8], axis=0)

    row2 = edge_index[0].reshape(NW * 16, CH)
    col2 = edge_index[1].reshape(NW * 16, CH)

    g, expsc = _k1(x, W, acat, r16)

    m_ref = jax.new_ref(jnp.zeros((N_NODES * N_NODES,), jnp.float32))
    val, rid = _k3_scatter_csr(row2, col2, g, m_ref)
    M = m_ref[...].reshape(N_NODES, N_NODES)
    agg = _k5(val, rid.reshape(NRANGE, 1, NW * CAP))

    es = expsc[:, :HEADS]
    ed1 = jnp.concatenate(
        [expsc[:, HEADS:], jnp.ones((N_NODES, 1), jnp.float32),
         jnp.zeros((N_NODES, 7), jnp.float32)], axis=1)

    return _k4(M, ed1, es, agg, r8)


# speedup vs baseline: 54.9398x; 1.0505x over previous
import functools

import jax
import jax.numpy as jnp
from jax import lax
from jax.experimental import pallas as pl
from jax.experimental.pallas import tpu as pltpu
from jax.experimental.pallas import tpu_sc as plsc

N_NODES = 2048
N_EDGES = 65536
IN_FEATURES = 256
HEADS = 8
OUT_FEATURES = 32
HF = HEADS * OUT_FEATURES

NC = 2
NS = 16
NW = NC * NS
EPW = N_EDGES // NW
CH = 128
NCH = EPW // CH

ROW_BLK = 256
GRID = N_NODES // ROW_BLK

_PREC = lax.Precision.HIGHEST


def _k1_body(x_ref, w_ref, acat_ref, r16_ref, g_ref, e_ref):
    h = lax.dot_general(x_ref[...], w_ref[...], (((1,), (0,)), ((), ())),
                        preferred_element_type=jnp.float32)
    s = lax.dot_general(h, acat_ref[...], (((1,), (0,)), ((), ())),
                        preferred_element_type=jnp.float32, precision=_PREC)
    e = jnp.exp(s)
    e_ref[...] = e
    edfull = lax.dot_general(e, r16_ref[...], (((1,), (0,)), ((), ())),
                             preferred_element_type=jnp.float32, precision=_PREC)
    g_ref[...] = h * edfull


_k1 = pl.pallas_call(
    _k1_body,
    grid=(GRID,),
    in_specs=[
        pl.BlockSpec((ROW_BLK, IN_FEATURES), lambda i: (i, 0)),
        pl.BlockSpec((IN_FEATURES, HF), lambda i: (0, 0)),
        pl.BlockSpec((HF, 16), lambda i: (0, 0)),
        pl.BlockSpec((16, HF), lambda i: (0, 0)),
    ],
    out_specs=[
        pl.BlockSpec((ROW_BLK, HF), lambda i: (i, 0)),
        pl.BlockSpec((ROW_BLK, 16), lambda i: (i, 0)),
    ],
    out_shape=[
        jax.ShapeDtypeStruct((N_NODES, HF), jnp.float32),
        jax.ShapeDtypeStruct((N_NODES, 16), jnp.float32),
    ],
    compiler_params=pltpu.CompilerParams(dimension_semantics=("parallel",)),
)


_sc_mesh = plsc.VectorSubcoreMesh(core_axis_name="c", subcore_axis_name="s")


@functools.partial(
    pl.kernel, mesh=_sc_mesh, out_type=(),
    scratch_types=[
        pltpu.VMEM((16, CH), jnp.int32),
        pltpu.VMEM((16, CH), jnp.int32),
        pltpu.VMEM((16, CH), jnp.int32),
        pltpu.VMEM((CH,), jnp.float32),
        pltpu.SemaphoreType.DMA,
    ],
    name="sc_scatter_mask",
)
def _k2_scatter_mask(row2_hbm, col2_hbm, m_ref, rowv, colv, keyv, onesv, sem):
    wid = lax.axis_index("s") * NC + lax.axis_index("c")
    pltpu.sync_copy(row2_hbm.at[pl.ds(wid * 16, 16)], rowv)
    pltpu.sync_copy(col2_hbm.at[pl.ds(wid * 16, 16)], colv)
    for l in range(CH // 16):
        onesv[pl.ds(l * 16, 16)] = jnp.ones((16,), jnp.float32)
    for j in range(NCH):
        for l in range(CH // 16):
            sl = pl.ds(l * 16, 16)
            keyv[j, sl] = rowv[j, sl] * N_NODES + colv[j, sl]
    copies = [pltpu.async_copy(onesv, m_ref.at[keyv.at[j]], sem)
              for j in range(NCH)]
    for cp in copies:
        cp.wait()


RPW = N_NODES // NW
NRANGE = NW
CAP = 128
SLOTS = NRANGE * NW * CAP
SENT = RPW


@functools.partial(
    pl.kernel, mesh=_sc_mesh,
    out_type=(
        jax.ShapeDtypeStruct((SLOTS, HF), jnp.float32),
        jax.ShapeDtypeStruct((SLOTS,), jnp.int32),
    ),
    scratch_types=[
        pltpu.VMEM((16, CH), jnp.int32),
        pltpu.VMEM((16, CH), jnp.int32),
        pltpu.VMEM((16, CH), jnp.int32),
        pltpu.VMEM((16, CH), jnp.int32),
        pltpu.VMEM((16, CH), jnp.int32),
        pltpu.VMEM((CH,), jnp.float32),
        pltpu.VMEM((NRANGE,), jnp.int32),
        pltpu.VMEM((CH,), jnp.int32),
        pltpu.VMEM((3, CH, HF), jnp.float32),
        pltpu.SemaphoreType.DMA,
        pltpu.SemaphoreType.DMA,
        pltpu.SemaphoreType.DMA,
    ],
    name="sc_scatter_csr",
    compiler_params=pltpu.CompilerParams(needs_layout_passes=False),
)
def _k3_scatter_csr(row2_hbm, col2_hbm, g_hbm, m_ref, val_ref, rid_ref,
                    rowv, colv, slotv, rlv, keyv, onesv, counters, sentv,
                    rows_v, sem_g, sem_s, sem_r):
    cid = lax.axis_index("c")
    sid = lax.axis_index("s")
    wid = sid * NC + cid
    pltpu.sync_copy(row2_hbm.at[pl.ds(wid * 16, 16)], rowv)
    pltpu.sync_copy(col2_hbm.at[pl.ds(wid * 16, 16)], colv)
    for l in range(NRANGE // 16):
        counters[pl.ds(l * 16, 16)] = jnp.zeros((16,), jnp.int32)
    for l in range(CH // 16):
        sentv[pl.ds(l * 16, 16)] = jnp.full((16,), SENT, jnp.int32)
    sent_cps = [pltpu.async_copy(
        sentv, rid_ref.at[pl.ds(rng * (NW * CAP) + wid * CAP, CAP)], sem_r)
        for rng in range(NRANGE)]
    for j in range(NCH):
        for l in range(CH // 16):
            sl = pl.ds(l * 16, 16)
            r16 = rowv[j, sl]
            rng16 = lax.shift_right_logical(r16, 6)
            base16 = plsc.load_gather(counters, [rng16])
            off16, last16 = plsc.scan_count(rng16)
            rank16 = base16 + off16 - 1
            plsc.store_scatter(counters, [rng16], rank16 + 1, mask=last16)
            rankc = jnp.minimum(rank16, CAP - 1)
            slotv[j, sl] = rng16 * (NW * CAP) + wid * CAP + rankc
            rlv[j, sl] = r16 - rng16 * RPW
            keyv[j, sl] = r16 * N_NODES + colv[j, sl]
    for l in range(CH // 16):
        onesv[pl.ds(l * 16, 16)] = jnp.ones((16,), jnp.float32)
    for cp in sent_cps:
        cp.wait()
    rid_cps = [pltpu.async_copy(rlv.at[j], rid_ref.at[slotv.at[j]], sem_r)
               for j in range(NCH)]
    m_cps = [pltpu.async_copy(onesv, m_ref.at[keyv.at[j]], sem_r)
             for j in range(NCH)]
    gcps, scps = [], []
    gcps.append(pltpu.async_copy(
        g_hbm.at[colv.at[0]], rows_v.at[0], sem_g))
    for j in range(NCH):
        nxt = j + 1
        if nxt < NCH:
            if nxt >= 3:
                scps[nxt - 3].wait()
            gcps.append(pltpu.async_copy(
                g_hbm.at[colv.at[nxt]], rows_v.at[nxt % 3], sem_g))
        gcps[j].wait()
        scps.append(pltpu.async_copy(
            rows_v.at[j % 3], val_ref.at[slotv.at[j]], sem_s))
    for j in range(NCH - 3, NCH):
        scps[j].wait()
    for cp in rid_cps:
        cp.wait()
    for cp in m_cps:
        cp.wait()


def _k5_body(val_ref, ridl_ref, out_ref):
    i = pl.program_id(0)
    ridl = ridl_ref[pl.ds(lax.rem(i, 8), 1), :]
    v = val_ref[...]
    vm = jnp.where(jnp.abs(v) < 3e38, v, 0.0)
    iot = lax.broadcasted_iota(jnp.int32, (RPW, NW * CAP), 0)
    p = jnp.where(iot == ridl, 1.0, 0.0)
    out_ref[...] = lax.dot_general(
        p, vm, (((1,), (0,)), ((), ())),
        preferred_element_type=jnp.float32)


_k5 = pl.pallas_call(
    _k5_body,
    grid=(NRANGE,),
    in_specs=[
        pl.BlockSpec((NW * CAP, HF), lambda i: (i, 0)),
        pl.BlockSpec((8, NW * CAP), lambda i: (i // 8, 0)),
    ],
    out_specs=pl.BlockSpec((RPW, HF), lambda i: (i, 0)),
    out_shape=jax.ShapeDtypeStruct((N_NODES, HF), jnp.float32),
    compiler_params=pltpu.CompilerParams(dimension_semantics=("parallel",)),
)


def _k4_body(m_ref, ed1_ref, es_ref, agg_ref, r8_ref, out_ref):
    td = lax.dot_general(m_ref[...], ed1_ref[...], (((1,), (0,)), ((), ())),
                         preferred_element_type=jnp.float32)
    t = td[:, 0:8]
    d = td[:, 8:9]
    esv = es_ref[...]
    denom = esv * t + (float(N_NODES) - d)
    scale = esv / denom
    sf = lax.dot_general(scale, r8_ref[...], (((1,), (0,)), ((), ())),
                         preferred_element_type=jnp.float32, precision=_PREC)
    out_ref[...] = jnp.maximum(agg_ref[...] * sf, 0.0)


_k4 = pl.pallas_call(
    _k4_body,
    grid=(GRID,),
    in_specs=[
        pl.BlockSpec((ROW_BLK, N_NODES), lambda i: (i, 0)),
        pl.BlockSpec((N_NODES, 16), lambda i: (0, 0)),
        pl.BlockSpec((ROW_BLK, HEADS), lambda i: (i, 0)),
        pl.BlockSpec((ROW_BLK, HF), lambda i: (i, 0)),
        pl.BlockSpec((HEADS, HF), lambda i: (0, 0)),
    ],
    out_specs=pl.BlockSpec((ROW_BLK, HF), lambda i: (i, 0)),
    out_shape=jax.ShapeDtypeStruct((N_NODES, HF), jnp.float32),
    compiler_params=pltpu.CompilerParams(dimension_semantics=("parallel",)),
)


def kernel(x, edge_index, W, a):
    a0 = a[0]
    asrc = a0[:, :OUT_FEATURES]
    adst = a0[:, OUT_FEATURES:]
    eye8 = jnp.eye(HEADS, dtype=jnp.float32)
    acat = jnp.concatenate(
        [(eye8[:, None, :] * asrc[:, :, None]).reshape(HF, HEADS),
         (eye8[:, None, :] * adst[:, :, None]).reshape(HF, HEADS)], axis=1)
    r8 = jnp.repeat(eye8, OUT_FEATURES, axis=1)
    r16 = jnp.concatenate([jnp.zeros((8, HF), jnp.float32), r8], axis=0)

    row2 = edge_index[0].reshape(NW * 16, CH)
    col2 = edge_index[1].reshape(NW * 16, CH)

    g, expsc = _k1(x, W, acat, r16)

    m_ref = jax.new_ref(jnp.zeros((N_NODES * N_NODES,), jnp.float32))
    val, rid = _k3_scatter_csr(row2, col2, g, m_ref)
    M = m_ref[...].reshape(N_NODES, N_NODES)
    agg = _k5(val, rid.reshape(NRANGE, NW * CAP))

    es = expsc[:, :HEADS]
    ed1 = jnp.concatenate(
        [expsc[:, HEADS:], jnp.ones((N_NODES, 1), jnp.float32),
         jnp.zeros((N_NODES, 7), jnp.float32)], axis=1)

    return _k4(M, ed1, es, agg, r8)


# speedup vs baseline: 72.9427x; 1.3277x over previous
import functools

import jax
import jax.numpy as jnp
from jax import lax
from jax.experimental import pallas as pl
from jax.experimental.pallas import tpu as pltpu
from jax.experimental.pallas import tpu_sc as plsc

N_NODES = 2048
N_EDGES = 65536
IN_FEATURES = 256
HEADS = 8
OUT_FEATURES = 32
HF = HEADS * OUT_FEATURES

NC = 2
NS = 16
NW = NC * NS
EPW = N_EDGES // NW
CH = 128
NCH = EPW // CH

ROW_BLK = 256
GRID = N_NODES // ROW_BLK

_PREC = lax.Precision.HIGHEST


def _k1_body(x_ref, w_ref, acat_ref, r16_ref, g_ref, e_ref):
    h = lax.dot_general(x_ref[...], w_ref[...], (((1,), (0,)), ((), ())),
                        preferred_element_type=jnp.float32)
    s = lax.dot_general(h, acat_ref[...], (((1,), (0,)), ((), ())),
                        preferred_element_type=jnp.float32, precision=_PREC)
    e = jnp.exp(s)
    e_ref[...] = e
    edfull = lax.dot_general(e, r16_ref[...], (((1,), (0,)), ((), ())),
                             preferred_element_type=jnp.float32, precision=_PREC)
    g_ref[...] = h * edfull


_k1 = pl.pallas_call(
    _k1_body,
    grid=(GRID,),
    in_specs=[
        pl.BlockSpec((ROW_BLK, IN_FEATURES), lambda i: (i, 0)),
        pl.BlockSpec((IN_FEATURES, HF), lambda i: (0, 0)),
        pl.BlockSpec((HF, 16), lambda i: (0, 0)),
        pl.BlockSpec((16, HF), lambda i: (0, 0)),
    ],
    out_specs=[
        pl.BlockSpec((ROW_BLK, HF), lambda i: (i, 0)),
        pl.BlockSpec((ROW_BLK, 16), lambda i: (i, 0)),
    ],
    out_shape=[
        jax.ShapeDtypeStruct((N_NODES, HF), jnp.float32),
        jax.ShapeDtypeStruct((N_NODES, 16), jnp.float32),
    ],
    compiler_params=pltpu.CompilerParams(dimension_semantics=("parallel",)),
)


_sc_mesh = plsc.VectorSubcoreMesh(core_axis_name="c", subcore_axis_name="s")


@functools.partial(
    pl.kernel, mesh=_sc_mesh, out_type=(),
    scratch_types=[
        pltpu.VMEM((16, CH), jnp.int32),
        pltpu.VMEM((16, CH), jnp.int32),
        pltpu.VMEM((16, CH), jnp.int32),
        pltpu.VMEM((CH,), jnp.float32),
        pltpu.SemaphoreType.DMA,
    ],
    name="sc_scatter_mask",
)
def _k2_scatter_mask(row2_hbm, col2_hbm, m_ref, rowv, colv, keyv, onesv, sem):
    wid = lax.axis_index("s") * NC + lax.axis_index("c")
    pltpu.sync_copy(row2_hbm.at[pl.ds(wid * 16, 16)], rowv)
    pltpu.sync_copy(col2_hbm.at[pl.ds(wid * 16, 16)], colv)
    for l in range(CH // 16):
        onesv[pl.ds(l * 16, 16)] = jnp.ones((16,), jnp.float32)
    for j in range(NCH):
        for l in range(CH // 16):
            sl = pl.ds(l * 16, 16)
            keyv[j, sl] = rowv[j, sl] * N_NODES + colv[j, sl]
    copies = [pltpu.async_copy(onesv, m_ref.at[keyv.at[j]], sem)
              for j in range(NCH)]
    for cp in copies:
        cp.wait()


RPW = N_NODES // NW
NRANGE = NW
CAP = 128
SLOTS = NRANGE * NW * CAP
SENT = RPW


@functools.partial(
    pl.kernel, mesh=_sc_mesh,
    out_type=(
        jax.ShapeDtypeStruct((SLOTS, HF), jnp.float32),
        jax.ShapeDtypeStruct((NRANGE, NW * CAP), jnp.int32),
    ),
    scratch_types=[
        pltpu.VMEM((16, CH), jnp.int32),
        pltpu.VMEM((16, CH), jnp.int32),
        pltpu.VMEM((16, CH), jnp.int32),
        pltpu.VMEM((16, CH), jnp.int32),
        pltpu.VMEM((16, CH), jnp.int32),
        pltpu.VMEM((CH,), jnp.float32),
        pltpu.VMEM((NRANGE,), jnp.int32),
        pltpu.VMEM((NRANGE, CAP), jnp.int32),
        pltpu.VMEM((3, CH, HF), jnp.float32),
        pltpu.SemaphoreType.DMA,
        pltpu.SemaphoreType.DMA,
        pltpu.SemaphoreType.DMA,
    ],
    name="sc_scatter_csr",
    compiler_params=pltpu.CompilerParams(needs_layout_passes=False),
)
def _k3_scatter_csr(row2_hbm, col2_hbm, g_hbm, m_ref, val_ref, rid_ref,
                    rowv, colv, slotv, rlv, keyv, onesv, counters, ridbuf,
                    rows_v, sem_g, sem_s, sem_r):
    cid = lax.axis_index("c")
    sid = lax.axis_index("s")
    wid = sid * NC + cid
    pltpu.sync_copy(row2_hbm.at[pl.ds(wid * 16, 16)], rowv)
    pltpu.sync_copy(col2_hbm.at[pl.ds(wid * 16, 16)], colv)
    for l in range(NRANGE // 16):
        counters[pl.ds(l * 16, 16)] = jnp.zeros((16,), jnp.int32)
    for rng in range(NRANGE):
        for l in range(CAP // 16):
            ridbuf[rng, pl.ds(l * 16, 16)] = jnp.full((16,), SENT, jnp.int32)
    for j in range(NCH):
        for l in range(CH // 16):
            sl = pl.ds(l * 16, 16)
            r16 = rowv[j, sl]
            rng16 = lax.shift_right_logical(r16, 6)
            base16 = plsc.load_gather(counters, [rng16])
            off16, last16 = plsc.scan_count(rng16)
            rank16 = base16 + off16 - 1
            plsc.store_scatter(counters, [rng16], rank16 + 1, mask=last16)
            rankc = jnp.minimum(rank16, CAP - 1)
            slotv[j, sl] = rng16 * (NW * CAP) + wid * CAP + rankc
            plsc.store_scatter(ridbuf, [rng16, rankc], r16 - rng16 * RPW)
            keyv[j, sl] = r16 * N_NODES + colv[j, sl]
    for l in range(CH // 16):
        onesv[pl.ds(l * 16, 16)] = jnp.ones((16,), jnp.float32)
    rid_cps = [pltpu.async_copy(
        ridbuf, rid_ref.at[:, pl.ds(wid * CAP, CAP)], sem_r)]
    m_cps = [pltpu.async_copy(onesv, m_ref.at[keyv.at[j]], sem_r)
             for j in range(NCH)]
    gcps, scps = [], []
    gcps.append(pltpu.async_copy(
        g_hbm.at[colv.at[0]], rows_v.at[0], sem_g))
    for j in range(NCH):
        nxt = j + 1
        if nxt < NCH:
            if nxt >= 3:
                scps[nxt - 3].wait()
            gcps.append(pltpu.async_copy(
                g_hbm.at[colv.at[nxt]], rows_v.at[nxt % 3], sem_g))
        gcps[j].wait()
        scps.append(pltpu.async_copy(
            rows_v.at[j % 3], val_ref.at[slotv.at[j]], sem_s))
    for j in range(NCH - 3, NCH):
        scps[j].wait()
    for cp in rid_cps:
        cp.wait()
    for cp in m_cps:
        cp.wait()


def _k5_body(val_ref, ridl_ref, out_ref):
    i = pl.program_id(0)
    ridl = ridl_ref[pl.ds(lax.rem(i, 8), 1), :]
    v = val_ref[...]
    vm = jnp.where(jnp.abs(v) < 3e38, v, 0.0)
    iot = lax.broadcasted_iota(jnp.int32, (RPW, NW * CAP), 0)
    p = jnp.where(iot == ridl, 1.0, 0.0)
    out_ref[...] = lax.dot_general(
        p, vm, (((1,), (0,)), ((), ())),
        preferred_element_type=jnp.float32)


_k5 = pl.pallas_call(
    _k5_body,
    grid=(NRANGE,),
    in_specs=[
        pl.BlockSpec((NW * CAP, HF), lambda i: (i, 0)),
        pl.BlockSpec((8, NW * CAP), lambda i: (i // 8, 0)),
    ],
    out_specs=pl.BlockSpec((RPW, HF), lambda i: (i, 0)),
    out_shape=jax.ShapeDtypeStruct((N_NODES, HF), jnp.float32),
    compiler_params=pltpu.CompilerParams(dimension_semantics=("parallel",)),
)


def _k4_body(m_ref, ed1_ref, es_ref, agg_ref, r8_ref, out_ref):
    td = lax.dot_general(m_ref[...], ed1_ref[...], (((1,), (0,)), ((), ())),
                         preferred_element_type=jnp.float32)
    t = td[:, 0:8]
    d = td[:, 8:9]
    esv = es_ref[...]
    denom = esv * t + (float(N_NODES) - d)
    scale = esv / denom
    sf = lax.dot_general(scale, r8_ref[...], (((1,), (0,)), ((), ())),
                         preferred_element_type=jnp.float32, precision=_PREC)
    out_ref[...] = jnp.maximum(agg_ref[...] * sf, 0.0)


_k4 = pl.pallas_call(
    _k4_body,
    grid=(GRID,),
    in_specs=[
        pl.BlockSpec((ROW_BLK, N_NODES), lambda i: (i, 0)),
        pl.BlockSpec((N_NODES, 16), lambda i: (0, 0)),
        pl.BlockSpec((ROW_BLK, HEADS), lambda i: (i, 0)),
        pl.BlockSpec((ROW_BLK, HF), lambda i: (i, 0)),
        pl.BlockSpec((HEADS, HF), lambda i: (0, 0)),
    ],
    out_specs=pl.BlockSpec((ROW_BLK, HF), lambda i: (i, 0)),
    out_shape=jax.ShapeDtypeStruct((N_NODES, HF), jnp.float32),
    compiler_params=pltpu.CompilerParams(dimension_semantics=("parallel",)),
)


def kernel(x, edge_index, W, a):
    a0 = a[0]
    asrc = a0[:, :OUT_FEATURES]
    adst = a0[:, OUT_FEATURES:]
    eye8 = jnp.eye(HEADS, dtype=jnp.float32)
    acat = jnp.concatenate(
        [(eye8[:, None, :] * asrc[:, :, None]).reshape(HF, HEADS),
         (eye8[:, None, :] * adst[:, :, None]).reshape(HF, HEADS)], axis=1)
    r8 = jnp.repeat(eye8, OUT_FEATURES, axis=1)
    r16 = jnp.concatenate([jnp.zeros((8, HF), jnp.float32), r8], axis=0)

    row2 = edge_index[0].reshape(NW * 16, CH)
    col2 = edge_index[1].reshape(NW * 16, CH)

    g, expsc = _k1(x, W, acat, r16)

    m_ref = jax.new_ref(jnp.zeros((N_NODES * N_NODES,), jnp.float32))
    val, rid = _k3_scatter_csr(row2, col2, g, m_ref)
    M = m_ref[...].reshape(N_NODES, N_NODES)
    agg = _k5(val, rid)

    es = expsc[:, :HEADS]
    ed1 = jnp.concatenate(
        [expsc[:, HEADS:], jnp.ones((N_NODES, 1), jnp.float32),
         jnp.zeros((N_NODES, 7), jnp.float32)], axis=1)

    return _k4(M, ed1, es, agg, r8)
